# baseline (speedup 1.0000x reference)
import functools
import math

import jax
import jax.numpy as jnp
from jax import lax
from jax.experimental import pallas as pl
from jax.experimental.pallas import tpu as pltpu
from jax.experimental.pallas import tpu_sc as plsc

_EMBED = 128
_D = 2 * _EMBED
_NC, _NS, _L = 2, 16, 16
_NW = _NC * _NS
_C = 80
_NBUF = 2

_N_NODES_PAD = 10240
_N_EDGES = 320000
_NODE_PER_W = _N_NODES_PAD // _NW
_EDGE_PER_W = _N_EDGES // _NW
_NODE_CHUNKS = _NODE_PER_W // _C
_EDGE_CHUNKS = _EDGE_PER_W // _C
_EDGE_GROUPS = (_EDGE_CHUNKS - 1) // _NBUF

_ETAB_WORDS = 20 * _D
_ATAB_WORDS = 100 * _EMBED
_CTAB_WORDS = 13 * _EMBED


def _sc_body(edge_tab, atab, ctab, na, nb, ea, eb, node_out, edge_out,
             et_v, at_v, ct_v, idx_e, tmp, idx_n, r0, r1, w0, w1):
  rows = (r0, r1)
  wsem = (w0, w1)
  wid = lax.axis_index("s") * _NC + lax.axis_index("c")
  ebase = wid * _EDGE_PER_W
  nbase = wid * _NODE_PER_W

  pltpu.sync_copy(edge_tab, et_v)
  pltpu.sync_copy(atab, at_v)
  pltpu.sync_copy(ctab, ct_v)

  pltpu.sync_copy(ea.at[pl.ds(ebase, _EDGE_PER_W)],
                  idx_e.at[pl.ds(0, _EDGE_PER_W)])
  pltpu.sync_copy(eb.at[pl.ds(ebase, _EDGE_PER_W)],
                  tmp.at[pl.ds(0, _EDGE_PER_W)])

  @plsc.parallel_loop(0, _EDGE_PER_W // _L, unroll=4)
  def _e_idx(i):
    sl = pl.ds(i * _L, _L)
    idx_e[sl] = (idx_e[sl] * 2 + tmp[sl]) * _D

  pltpu.sync_copy(na.at[pl.ds(nbase, _NODE_PER_W)],
                  idx_n.at[pl.ds(0, _NODE_PER_W)])
  pltpu.sync_copy(nb.at[pl.ds(nbase, _NODE_PER_W)],
                  tmp.at[pl.ds(0, _NODE_PER_W)])

  @plsc.parallel_loop(0, _NODE_PER_W // _L, unroll=4)
  def _n_idx(i):
    sl = pl.ds(i * _L, _L)
    idx_n[sl] = idx_n[sl] * _EMBED
    tmp[sl] = (tmp[sl] + 7) * _EMBED

  def e_write(j, b):
    pltpu.async_copy(rows[b], edge_out.at[pl.ds(ebase + j * _C, _C)], wsem[b])

  def e_write_wait(b):
    pltpu.make_async_copy(rows[b], edge_out.at[pl.ds(ebase, _C)],
                          wsem[b]).wait()

  def e_assemble(j, b):
    buf = rows[b]
    @plsc.parallel_loop(0, _C, step=4)
    def _row(r):
      vv = idx_e[pl.ds(j * _C + r, _L)]
      for l in range(4):
        base = vv[l]
        for k in range(_D // _L):
          buf[r + l, pl.ds(k * _L, _L)] = et_v[pl.ds(base + k * _L, _L)]

  def e_group(g, _):
    for b in range(_NBUF):
      @pl.when(g > 0)
      def _():
        e_write_wait(b)
      e_assemble(g * _NBUF + b, b)
      e_write(g * _NBUF + b, b)
    return 0
  lax.fori_loop(0, _EDGE_GROUPS, e_group, 0)
  e_write_wait(0)
  e_assemble(_EDGE_CHUNKS - 1, 0)
  e_write(_EDGE_CHUNKS - 1, 0)
  e_write_wait(0)
  e_write_wait(1)

  def n_write(j, b):
    pltpu.async_copy(rows[b], node_out.at[pl.ds(nbase + j * _C, _C)], wsem[b])

  def n_write_wait(b):
    pltpu.make_async_copy(rows[b], node_out.at[pl.ds(nbase, _C)],
                          wsem[b]).wait()

  def n_assemble(j, b):
    buf = rows[b]
    @plsc.parallel_loop(0, _C, step=4)
    def _row(r):
      va = idx_n[pl.ds(j * _C + r, _L)]
      vc = tmp[pl.ds(j * _C + r, _L)]
      for l in range(4):
        base_a = va[l]
        base_c = vc[l]
        for k in range(_EMBED // _L):
          buf[r + l, pl.ds(k * _L, _L)] = at_v[pl.ds(base_a + k * _L, _L)]
        for k in range(_EMBED // _L):
          buf[r + l, pl.ds(_EMBED + k * _L, _L)] = ct_v[pl.ds(base_c + k * _L, _L)]

  for j in range(_NODE_CHUNKS):
    b = j % _NBUF
    if j >= _NBUF:
      n_write_wait(b)
    n_assemble(j, b)
    n_write(j, b)
  n_write_wait(0)
  n_write_wait(1)


@functools.partial(
    pl.kernel,
    out_type=(
        jax.ShapeDtypeStruct((_N_NODES_PAD, _D), jnp.float32),
        jax.ShapeDtypeStruct((_N_EDGES, _D), jnp.float32),
    ),
    mesh=plsc.VectorSubcoreMesh(core_axis_name="c", subcore_axis_name="s"),
    scratch_types=[
        pltpu.VMEM((_ETAB_WORDS,), jnp.float32),
        pltpu.VMEM((_ATAB_WORDS,), jnp.float32),
        pltpu.VMEM((_CTAB_WORDS,), jnp.float32),
        pltpu.VMEM((_EDGE_PER_W + _L,), jnp.int32),
        pltpu.VMEM((_EDGE_PER_W + _L,), jnp.int32),
        pltpu.VMEM((_NODE_PER_W + _L,), jnp.int32),
        pltpu.VMEM((_C, _D), jnp.float32),
        pltpu.VMEM((_C, _D), jnp.float32),
        pltpu.SemaphoreType.DMA,
        pltpu.SemaphoreType.DMA,
    ],
)
def _sc_embed(edge_tab, atab, ctab, na, nb, ea, eb, node_out, edge_out,
              et_v, at_v, ct_v, idx_e, tmp, idx_n, r0, r1, w0, w1):
  _sc_body(edge_tab, atab, ctab, na, nb, ea, eb, node_out, edge_out,
           et_v, at_v, ct_v, idx_e, tmp, idx_n, r0, r1, w0, w1)


def kernel(graph_x, graph_edge_attr, atomic_table, charge_table, bond_table,
           arom_table):
  num_bonds = bond_table.shape[0]
  num_arom = arom_table.shape[0]

  edge_tab = jnp.concatenate(
      (jnp.repeat(bond_table, num_arom, axis=0),
       jnp.tile(arom_table, (num_bonds, 1))), axis=1)

  n_nodes = graph_x.shape[0]
  na = jnp.pad(graph_x[:, 0].astype(jnp.int32), (0, _N_NODES_PAD - n_nodes))
  nb = jnp.pad(graph_x[:, 1].astype(jnp.int32), (0, _N_NODES_PAD - n_nodes))
  ea = graph_edge_attr[:, 0].astype(jnp.int32)
  eb = graph_edge_attr[:, 1].astype(jnp.int32)

  node_out, edge_out = _sc_embed(
      edge_tab.reshape(-1), atomic_table.reshape(-1),
      charge_table.reshape(-1), na, nb, ea, eb)
  return node_out[:n_nodes], edge_out


# speedup vs baseline: 1.1703x; 1.1703x over previous
import functools
import math

import jax
import jax.numpy as jnp
from jax import lax
from jax.experimental import pallas as pl
from jax.experimental.pallas import tpu as pltpu
from jax.experimental.pallas import tpu_sc as plsc

_EMBED = 128
_D = 2 * _EMBED
_NC, _NS, _L = 2, 16, 16
_NW = _NC * _NS
_C = 80
_NBUF = 2

_N_NODES_PAD = 10240
_N_EDGES = 320000
_NODE_PER_W = _N_NODES_PAD // _NW
_EDGE_PER_W = _N_EDGES // _NW
_NODE_CHUNKS = _NODE_PER_W // _C
_EDGE_CHUNKS = _EDGE_PER_W // _C
_EDGE_GROUPS = (_EDGE_CHUNKS - 1) // _NBUF

_ETAB_WORDS = 20 * _D
_ATAB_WORDS = 100 * _EMBED
_CTAB_WORDS = 13 * _EMBED


def _sc_body(edge_tab, atab, ctab, na, nb, ea, eb, node_out, edge_out,
             et_v, at_v, ct_v, idx_e, tmp, idx_n, r0, r1, w0, w1):
  rows = (r0, r1)
  wsem = (w0, w1)
  wid = lax.axis_index("s") * _NC + lax.axis_index("c")
  ebase = wid * _EDGE_PER_W
  nbase = wid * _NODE_PER_W

  pltpu.sync_copy(edge_tab, et_v)
  pltpu.sync_copy(atab, at_v)
  pltpu.sync_copy(ctab, ct_v)

  pltpu.sync_copy(ea.at[pl.ds(ebase, _EDGE_PER_W)],
                  idx_e.at[pl.ds(0, _EDGE_PER_W)])
  pltpu.sync_copy(eb.at[pl.ds(ebase, _EDGE_PER_W)],
                  tmp.at[pl.ds(0, _EDGE_PER_W)])

  @plsc.parallel_loop(0, _EDGE_PER_W // _L, unroll=4)
  def _e_idx(i):
    sl = pl.ds(i * _L, _L)
    idx_e[sl] = (idx_e[sl] * 2 + tmp[sl]) * _D

  pltpu.sync_copy(na.at[pl.ds(nbase, _NODE_PER_W)],
                  idx_n.at[pl.ds(0, _NODE_PER_W)])
  pltpu.sync_copy(nb.at[pl.ds(nbase, _NODE_PER_W)],
                  tmp.at[pl.ds(0, _NODE_PER_W)])

  @plsc.parallel_loop(0, _NODE_PER_W // _L, unroll=4)
  def _n_idx(i):
    sl = pl.ds(i * _L, _L)
    idx_n[sl] = idx_n[sl] * _EMBED
    tmp[sl] = (tmp[sl] + 7) * _EMBED

  def e_write(j, b):
    pltpu.async_copy(rows[b], edge_out.at[pl.ds(ebase + j * _C, _C)], wsem[b])

  def e_write_wait(b):
    pltpu.make_async_copy(rows[b], edge_out.at[pl.ds(ebase, _C)],
                          wsem[b]).wait()

  def e_assemble(j, b):
    buf = rows[b]
    @plsc.parallel_loop(0, _C, unroll=4)
    def _row(r):
      base = idx_e[pl.ds(j * _C + r, _L)][0]
      for k in range(_D // _L):
        buf[r, pl.ds(k * _L, _L)] = et_v[pl.ds(base + k * _L, _L)]

  def e_group(g, _):
    for b in range(_NBUF):
      @pl.when(g > 0)
      def _():
        e_write_wait(b)
      e_assemble(g * _NBUF + b, b)
      e_write(g * _NBUF + b, b)
    return 0
  lax.fori_loop(0, _EDGE_GROUPS, e_group, 0)
  e_write_wait(0)
  e_assemble(_EDGE_CHUNKS - 1, 0)
  e_write(_EDGE_CHUNKS - 1, 0)
  e_write_wait(0)
  e_write_wait(1)

  def n_write(j, b):
    pltpu.async_copy(rows[b], node_out.at[pl.ds(nbase + j * _C, _C)], wsem[b])

  def n_write_wait(b):
    pltpu.make_async_copy(rows[b], node_out.at[pl.ds(nbase, _C)],
                          wsem[b]).wait()

  def n_assemble(j, b):
    buf = rows[b]
    @plsc.parallel_loop(0, _C, unroll=4)
    def _row(r):
      base_a = idx_n[pl.ds(j * _C + r, _L)][0]
      base_c = tmp[pl.ds(j * _C + r, _L)][0]
      for k in range(_EMBED // _L):
        buf[r, pl.ds(k * _L, _L)] = at_v[pl.ds(base_a + k * _L, _L)]
      for k in range(_EMBED // _L):
        buf[r, pl.ds(_EMBED + k * _L, _L)] = ct_v[pl.ds(base_c + k * _L, _L)]

  for j in range(_NODE_CHUNKS):
    b = j % _NBUF
    if j >= _NBUF:
      n_write_wait(b)
    n_assemble(j, b)
    n_write(j, b)
  n_write_wait(0)
  n_write_wait(1)


@functools.partial(
    pl.kernel,
    out_type=(
        jax.ShapeDtypeStruct((_N_NODES_PAD, _D), jnp.float32),
        jax.ShapeDtypeStruct((_N_EDGES, _D), jnp.float32),
    ),
    mesh=plsc.VectorSubcoreMesh(core_axis_name="c", subcore_axis_name="s"),
    scratch_types=[
        pltpu.VMEM((_ETAB_WORDS,), jnp.float32),
        pltpu.VMEM((_ATAB_WORDS,), jnp.float32),
        pltpu.VMEM((_CTAB_WORDS,), jnp.float32),
        pltpu.VMEM((_EDGE_PER_W + _L,), jnp.int32),
        pltpu.VMEM((_EDGE_PER_W + _L,), jnp.int32),
        pltpu.VMEM((_NODE_PER_W + _L,), jnp.int32),
        pltpu.VMEM((_C, _D), jnp.float32),
        pltpu.VMEM((_C, _D), jnp.float32),
        pltpu.SemaphoreType.DMA,
        pltpu.SemaphoreType.DMA,
    ],
)
def _sc_embed(edge_tab, atab, ctab, na, nb, ea, eb, node_out, edge_out,
              et_v, at_v, ct_v, idx_e, tmp, idx_n, r0, r1, w0, w1):
  _sc_body(edge_tab, atab, ctab, na, nb, ea, eb, node_out, edge_out,
           et_v, at_v, ct_v, idx_e, tmp, idx_n, r0, r1, w0, w1)


def kernel(graph_x, graph_edge_attr, atomic_table, charge_table, bond_table,
           arom_table):
  num_bonds = bond_table.shape[0]
  num_arom = arom_table.shape[0]

  edge_tab = jnp.concatenate(
      (jnp.repeat(bond_table, num_arom, axis=0),
       jnp.tile(arom_table, (num_bonds, 1))), axis=1)

  n_nodes = graph_x.shape[0]
  na = jnp.pad(graph_x[:, 0].astype(jnp.int32), (0, _N_NODES_PAD - n_nodes))
  nb = jnp.pad(graph_x[:, 1].astype(jnp.int32), (0, _N_NODES_PAD - n_nodes))
  ea = graph_edge_attr[:, 0].astype(jnp.int32)
  eb = graph_edge_attr[:, 1].astype(jnp.int32)

  node_out, edge_out = _sc_embed(
      edge_tab.reshape(-1), atomic_table.reshape(-1),
      charge_table.reshape(-1), na, nb, ea, eb)
  return node_out[:n_nodes], edge_out


# speedup vs baseline: 1.1889x; 1.0158x over previous
import functools
import math

import jax
import jax.numpy as jnp
from jax import lax
from jax.experimental import pallas as pl
from jax.experimental.pallas import tpu as pltpu
from jax.experimental.pallas import tpu_sc as plsc

_EMBED = 128
_D = 2 * _EMBED
_NC, _NS, _L = 2, 16, 16
_NW = _NC * _NS
_C = 80
_NBUF = 2

_N_NODES_PAD = 10240
_N_EDGES = 320000
_NODE_PER_W = _N_NODES_PAD // _NW
_EDGE_PER_W = _N_EDGES // _NW
_NODE_CHUNKS = _NODE_PER_W // _C
_EDGE_CHUNKS = _EDGE_PER_W // _C
_EDGE_GROUPS = (_EDGE_CHUNKS - 1) // _NBUF

_ETAB_WORDS = 20 * _D
_ATAB_WORDS = 100 * _EMBED
_CTAB_WORDS = 13 * _EMBED


def _sc_body(edge_tab, atab, ctab, na, nb, ea, eb, node_out, edge_out,
             et_v, at_v, ct_v, idx_e, tmp, idx_n, r0, r1, w0, w1):
  rows = (r0, r1)
  wsem = (w0, w1)
  wid = lax.axis_index("s") * _NC + lax.axis_index("c")
  ebase = wid * _EDGE_PER_W
  nbase = wid * _NODE_PER_W

  pltpu.sync_copy(edge_tab, et_v)
  pltpu.sync_copy(atab, at_v)
  pltpu.sync_copy(ctab, ct_v)

  pltpu.sync_copy(ea.at[pl.ds(ebase, _EDGE_PER_W)],
                  idx_e.at[pl.ds(0, _EDGE_PER_W)])
  pltpu.sync_copy(eb.at[pl.ds(ebase, _EDGE_PER_W)],
                  tmp.at[pl.ds(0, _EDGE_PER_W)])

  @plsc.parallel_loop(0, _EDGE_PER_W // _L, unroll=4)
  def _e_idx(i):
    sl = pl.ds(i * _L, _L)
    idx_e[sl] = (idx_e[sl] * 2 + tmp[sl]) * _D

  pltpu.sync_copy(na.at[pl.ds(nbase, _NODE_PER_W)],
                  idx_n.at[pl.ds(0, _NODE_PER_W)])
  pltpu.sync_copy(nb.at[pl.ds(nbase, _NODE_PER_W)],
                  tmp.at[pl.ds(0, _NODE_PER_W)])

  @plsc.parallel_loop(0, _NODE_PER_W // _L, unroll=4)
  def _n_idx(i):
    sl = pl.ds(i * _L, _L)
    idx_n[sl] = idx_n[sl] * _EMBED
    tmp[sl] = (tmp[sl] + 7) * _EMBED

  def e_write(j, b):
    pltpu.async_copy(rows[b], edge_out.at[pl.ds(ebase + j * _C, _C)], wsem[b])

  def e_write_wait(b):
    pltpu.make_async_copy(rows[b], edge_out.at[pl.ds(ebase, _C)],
                          wsem[b]).wait()

  def e_assemble(j, b):
    buf = rows[b]
    @plsc.parallel_loop(0, _C, unroll=2)
    def _row(r):
      base = idx_e[pl.ds(j * _C + r, _L)][0]
      for k in range(_D // _L):
        buf[r, pl.ds(k * _L, _L)] = et_v[pl.ds(base + k * _L, _L)]

  def e_group(g, _):
    for b in range(_NBUF):
      @pl.when(g > 0)
      def _():
        e_write_wait(b)
      e_assemble(g * _NBUF + b, b)
      e_write(g * _NBUF + b, b)
    return 0
  lax.fori_loop(0, _EDGE_GROUPS, e_group, 0)
  e_write_wait(0)
  e_assemble(_EDGE_CHUNKS - 1, 0)
  e_write(_EDGE_CHUNKS - 1, 0)
  e_write_wait(0)
  e_write_wait(1)

  def n_write(j, b):
    pltpu.async_copy(rows[b], node_out.at[pl.ds(nbase + j * _C, _C)], wsem[b])

  def n_write_wait(b):
    pltpu.make_async_copy(rows[b], node_out.at[pl.ds(nbase, _C)],
                          wsem[b]).wait()

  def n_assemble(j, b):
    buf = rows[b]
    @plsc.parallel_loop(0, _C, unroll=2)
    def _row(r):
      base_a = idx_n[pl.ds(j * _C + r, _L)][0]
      base_c = tmp[pl.ds(j * _C + r, _L)][0]
      for k in range(_EMBED // _L):
        buf[r, pl.ds(k * _L, _L)] = at_v[pl.ds(base_a + k * _L, _L)]
      for k in range(_EMBED // _L):
        buf[r, pl.ds(_EMBED + k * _L, _L)] = ct_v[pl.ds(base_c + k * _L, _L)]

  for j in range(_NODE_CHUNKS):
    b = j % _NBUF
    if j >= _NBUF:
      n_write_wait(b)
    n_assemble(j, b)
    n_write(j, b)
  n_write_wait(0)
  n_write_wait(1)


@functools.partial(
    pl.kernel,
    out_type=(
        jax.ShapeDtypeStruct((_N_NODES_PAD, _D), jnp.float32),
        jax.ShapeDtypeStruct((_N_EDGES, _D), jnp.float32),
    ),
    mesh=plsc.VectorSubcoreMesh(core_axis_name="c", subcore_axis_name="s"),
    scratch_types=[
        pltpu.VMEM((_ETAB_WORDS,), jnp.float32),
        pltpu.VMEM((_ATAB_WORDS,), jnp.float32),
        pltpu.VMEM((_CTAB_WORDS,), jnp.float32),
        pltpu.VMEM((_EDGE_PER_W + _L,), jnp.int32),
        pltpu.VMEM((_EDGE_PER_W + _L,), jnp.int32),
        pltpu.VMEM((_NODE_PER_W + _L,), jnp.int32),
        pltpu.VMEM((_C, _D), jnp.float32),
        pltpu.VMEM((_C, _D), jnp.float32),
        pltpu.SemaphoreType.DMA,
        pltpu.SemaphoreType.DMA,
    ],
)
def _sc_embed(edge_tab, atab, ctab, na, nb, ea, eb, node_out, edge_out,
              et_v, at_v, ct_v, idx_e, tmp, idx_n, r0, r1, w0, w1):
  _sc_body(edge_tab, atab, ctab, na, nb, ea, eb, node_out, edge_out,
           et_v, at_v, ct_v, idx_e, tmp, idx_n, r0, r1, w0, w1)


def kernel(graph_x, graph_edge_attr, atomic_table, charge_table, bond_table,
           arom_table):
  num_bonds = bond_table.shape[0]
  num_arom = arom_table.shape[0]

  edge_tab = jnp.concatenate(
      (jnp.repeat(bond_table, num_arom, axis=0),
       jnp.tile(arom_table, (num_bonds, 1))), axis=1)

  n_nodes = graph_x.shape[0]
  na = jnp.pad(graph_x[:, 0].astype(jnp.int32), (0, _N_NODES_PAD - n_nodes))
  nb = jnp.pad(graph_x[:, 1].astype(jnp.int32), (0, _N_NODES_PAD - n_nodes))
  ea = graph_edge_attr[:, 0].astype(jnp.int32)
  eb = graph_edge_attr[:, 1].astype(jnp.int32)

  node_out, edge_out = _sc_embed(
      edge_tab.reshape(-1), atomic_table.reshape(-1),
      charge_table.reshape(-1), na, nb, ea, eb)
  return node_out[:n_nodes], edge_out
